# speedup vs baseline: 41.5936x; 41.5936x over previous
import jax
import jax.numpy as jnp
from jax.experimental import pallas as pl
from jax.experimental.pallas import tpu as pltpu

_CH = 2000
_LCH = 2048


def _level_ranges(n_full):
    levels = []
    l = 0
    while (2 ** l - 1) < n_full:
        lo = 2 ** l - 1
        hi = min(2 ** (l + 1) - 2, n_full - 1)
        levels.append((lo, hi))
        l += 1
    return list(reversed(levels))


def _tree_kernel(x_ref, h0_ref, c0_ref, wi_ref, bi_ref, uiou_ref, biou_ref,
                 uf_ref, ufb_ref, sw_ref, sb_ref, w_ref,
                 out_ref, h_s, c_s, iou_s):
    n, s = h0_ref.shape
    n_full = (n - 1) // 2
    levels = _level_ranges(n_full)

    for r in range(0, n, _CH):
        iou_s[r:r + _CH] = (
            jnp.dot(x_ref[r:r + _CH], wi_ref[:],
                    preferred_element_type=jnp.float32) + bi_ref[:])
    h_s[:] = h0_ref[:]
    c_s[:] = c0_ref[:]

    def prop():
        for r in range(n_full, n, _LCH):
            e = min(r + _LCH, n)
            iou_l = iou_s[r:e] + biou_ref[:]
            i_g = iou_l[:, :s]
            o_g = iou_l[:, s:2 * s]
            u_g = iou_l[:, 2 * s:]
            c_new = jax.nn.sigmoid(i_g) * jnp.tanh(u_g) + c_s[r:e]
            h_new = jax.nn.sigmoid(o_g) * jnp.tanh(c_new)
            h_s[r:e] = h_new
            c_s[r:e] = c_new
        for lo, hi in levels:
            m = hi - lo + 1
            a = 2 * lo + 1
            b = 2 * hi + 3
            hcat = h_s[a:b].reshape(m, 2 * s)
            ccat = c_s[a:b].reshape(m, 2 * s)
            f = jax.nn.sigmoid(
                jnp.dot(hcat, uf_ref[:], preferred_element_type=jnp.float32)
                + ufb_ref[:])
            c_red = f[:, :s] * ccat[:, :s] + f[:, s:] * ccat[:, s:]
            iou_n = jnp.dot(hcat, uiou_ref[:],
                            preferred_element_type=jnp.float32)
            ib = iou_n + biou_ref[:]
            c_new = jax.nn.sigmoid(ib[:, :s]) * jnp.tanh(ib[:, 2 * s:]) + c_red
            h_new = jax.nn.sigmoid(ib[:, s:2 * s]) * jnp.tanh(c_new)
            h_s[lo:hi + 1] = h_new
            c_s[lo:hi + 1] = c_new
            iou_s[lo:hi + 1] = iou_n

    def acc_out(k, first):
        w0 = w_ref[3 * k]
        w1 = w_ref[3 * k + 1]
        w2 = w_ref[3 * k + 2]
        for r in range(0, n, _CH):
            blk = iou_s[r:r + _CH]
            v = blk[:, :s] * w0 + blk[:, s:2 * s] * w1 + blk[:, 2 * s:] * w2
            if first:
                out_ref[r:r + _CH] = v
            else:
                out_ref[r:r + _CH] += v

    def stage(ix):
        for r in range(0, n, _CH):
            iou_s[r:r + _CH] = jnp.maximum(
                jnp.dot(iou_s[r:r + _CH], sw_ref[ix],
                        preferred_element_type=jnp.float32)
                + sb_ref[ix:ix + 1, :], 0.0)

    prop()
    acc_out(0, first=True)
    for ix in range(3):
        stage(ix)
        prop()
        acc_out(ix + 1, first=False)
    prop()
    for r in range(0, n, _CH):
        out_ref[r:r + _CH] += h_s[r:r + _CH] * w_ref[12] + w_ref[13]


def kernel(x, h, c, W_init, b_init, U_iou_w, b_iou, U_f_w, U_f_b,
           stage_W, stage_b, out_w, out_b):
    n, s = h.shape
    wvec = jnp.concatenate([out_w, out_b]).astype(jnp.float32)
    out = pl.pallas_call(
        _tree_kernel,
        out_shape=jax.ShapeDtypeStruct((n, s), jnp.float32),
        in_specs=[pl.BlockSpec(memory_space=pltpu.VMEM)] * 11
        + [pl.BlockSpec(memory_space=pltpu.SMEM)],
        out_specs=pl.BlockSpec(memory_space=pltpu.VMEM),
        scratch_shapes=[
            pltpu.VMEM((n, s), jnp.float32),
            pltpu.VMEM((n, s), jnp.float32),
            pltpu.VMEM((n, 3 * s), jnp.float32),
        ],
        compiler_params=pltpu.CompilerParams(
            vmem_limit_bytes=120 * 1024 * 1024),
    )(x, h, c,
      W_init.T, b_init.reshape(1, -1),
      U_iou_w.T, b_iou.reshape(1, -1),
      U_f_w.T, U_f_b.reshape(1, -1),
      jnp.transpose(stage_W, (0, 2, 1)), stage_b,
      wvec)
    return out.reshape(n, 1, 1, s)
